# baseline (speedup 1.0000x reference)
import functools

import jax
import jax.numpy as jnp
from jax import lax
from jax.experimental import pallas as pl
from jax.experimental.pallas import tpu as pltpu
from jax.experimental.pallas import tpu_sc as plsc

T = 50
D_TIME = 8
D_SPACE = 8
EMBED_DIM = 16
SPACER = 17
D_OUT = 2 * D_TIME + 2 + 2 * D_SPACE + SPACER + 1 + EMBED_DIM + 1

_NC = 2
_NS = 16
_NW = _NC * _NS


def _sc_embedding_gather(table, ids):
    B = ids.shape[0]
    D = table.shape[1]
    b_per_w = B // _NW
    mesh = plsc.VectorSubcoreMesh(core_axis_name="c", subcore_axis_name="s")

    @functools.partial(
        pl.kernel,
        mesh=mesh,
        out_type=jax.ShapeDtypeStruct((B, D), jnp.float32),
        scratch_types=[
            pltpu.VMEM((b_per_w,), jnp.int32),
            pltpu.VMEM((b_per_w, D), jnp.float32),
            pltpu.SemaphoreType.DMA,
        ],
    )
    def gather_kernel(table_hbm, idx_hbm, out_hbm, idx_v, rows_v, sem):
        wid = lax.axis_index("s") * _NC + lax.axis_index("c")
        base = wid * b_per_w
        pltpu.sync_copy(idx_hbm.at[pl.ds(base, b_per_w)], idx_v)
        pltpu.async_copy(table_hbm.at[idx_v], rows_v, sem).wait()
        pltpu.sync_copy(rows_v, out_hbm.at[pl.ds(base, b_per_w)])

    return gather_kernel(table, ids)


_SPLIT = 48


def _assemble_main_body(tf_ref, t0f_ref, az_ref, el_ref, y_ref, x_ref,
                        o_ref):
    bl = o_ref.shape[2]
    o_ref[:, 0:8, :] = tf_ref[...]
    o_ref[:, 8:16, :] = jnp.broadcast_to(t0f_ref[...][None], (T, D_TIME, bl))
    o_ref[:, 16:17, :] = jnp.reshape(az_ref[...], (T, 1, bl))
    o_ref[:, 17:18, :] = jnp.reshape(el_ref[...], (T, 1, bl))
    o_ref[:, 18:26, :] = jnp.broadcast_to(y_ref[...][None], (T, D_SPACE, bl))
    o_ref[:, 26:34, :] = jnp.broadcast_to(x_ref[...][None], (T, D_SPACE, bl))
    o_ref[:, 34:48, :] = jnp.zeros((T, _SPLIT - 34, bl), jnp.float32)


def _assemble_main(tf_t, t0f_t, az_t, el_t, y_t, x_t, block_l=512):
    B = tf_t.shape[2]
    grid = (B // block_l,)

    def m3(i):
        return (0, 0, i)

    def m2(i):
        return (0, i)

    return pl.pallas_call(
        _assemble_main_body,
        grid=grid,
        in_specs=[
            pl.BlockSpec((T, D_TIME, block_l), m3),
            pl.BlockSpec((D_TIME, block_l), m2),
            pl.BlockSpec((T, block_l), m2),
            pl.BlockSpec((T, block_l), m2),
            pl.BlockSpec((D_SPACE, block_l), m2),
            pl.BlockSpec((D_SPACE, block_l), m2),
        ],
        out_specs=pl.BlockSpec((T, _SPLIT, block_l), m3),
        out_shape=jax.ShapeDtypeStruct((T, D_OUT, B), jnp.float32),
    )(tf_t, t0f_t, az_t, el_t, y_t, x_t)


def _assemble_tail_body(prev_ref, t0i_ref, emb_ref, gsp_ref, o_ref):
    del prev_ref
    bl = o_ref.shape[2]
    o_ref[:, 0:3, :] = jnp.zeros((T, 3, bl), jnp.float32)
    o_ref[:, 3:4, :] = jnp.ones((T, 1, bl), jnp.float32)
    o_ref[:, 4:20, :] = jnp.broadcast_to(emb_ref[...][None],
                                         (T, EMBED_DIM, bl))
    t = lax.broadcasted_iota(jnp.int32, (T, 1, bl), 0)
    o_ref[:, 20:21, :] = jnp.where(t <= t0i_ref[0],
                                   gsp_ref[...], 0.0)


def _assemble_tail(prev, t0i, emb_t, gsp_t, block_l=512):
    B = gsp_t.shape[2]
    grid = (B // block_l,)

    def m2(i):
        return (0, i)

    return pl.pallas_call(
        _assemble_tail_body,
        grid=grid,
        in_specs=[
            pl.BlockSpec(memory_space=pl.ANY),
            pl.BlockSpec(memory_space=pltpu.SMEM),
            pl.BlockSpec((EMBED_DIM, block_l), m2),
            pl.BlockSpec((T, 1, block_l), lambda i: (0, 0, i)),
        ],
        out_specs=pl.BlockSpec(
            (pl.Element(T), pl.Element(24, padding=(0, 3)),
             pl.Element(block_l)),
            lambda i: (0, _SPLIT, i * block_l)),
        out_shape=jax.ShapeDtypeStruct((T, D_OUT, B), jnp.float32),
        input_output_aliases={0: 0},
    )(prev, t0i, emb_t, gsp_t)


def kernel(gsp, gsp_time_utc_fourier, gsp_solar_azimuth, gsp_solar_elevation,
           gsp_y_osgb_fourier, gsp_x_osgb_fourier, gsp_time_utc_fourier_t0,
           embedding_table, gsp_id, t0_idx):
    B = gsp.shape[0]
    ids = jnp.reshape(gsp_id, (B,))
    table_padded = jnp.pad(embedding_table,
                           ((0, 0), (0, 128 - embedding_table.shape[1])))
    emb = _sc_embedding_gather(table_padded, ids)
    emb_t = jnp.transpose(emb[:, 0:EMBED_DIM], (1, 0))
    t0i = jnp.reshape(jnp.asarray(t0_idx, jnp.int32), (1,))
    main_t = _assemble_main(
        jnp.transpose(gsp_time_utc_fourier, (1, 2, 0)),
        jnp.transpose(gsp_time_utc_fourier_t0, (1, 0)),
        jnp.transpose(gsp_solar_azimuth, (1, 0)),
        jnp.transpose(gsp_solar_elevation, (1, 0)),
        jnp.reshape(gsp_y_osgb_fourier, (B, D_SPACE)).T,
        jnp.reshape(gsp_x_osgb_fourier, (B, D_SPACE)).T,
    )
    out_t = _assemble_tail(
        main_t, t0i, emb_t,
        jnp.transpose(gsp, (1, 2, 0)),
    )
    return jnp.transpose(out_t, (2, 0, 1))


# speedup vs baseline: 1.0236x; 1.0236x over previous
import functools

import jax
import jax.numpy as jnp
from jax import lax
from jax.experimental import pallas as pl
from jax.experimental.pallas import tpu as pltpu
from jax.experimental.pallas import tpu_sc as plsc

T = 50
D_TIME = 8
D_SPACE = 8
EMBED_DIM = 16
SPACER = 17
D_OUT = 2 * D_TIME + 2 + 2 * D_SPACE + SPACER + 1 + EMBED_DIM + 1

_NC = 2
_NS = 16
_NW = _NC * _NS


def _sc_embedding_gather(table, ids):
    B = ids.shape[0]
    D = table.shape[1]
    b_per_w = B // _NW
    mesh = plsc.VectorSubcoreMesh(core_axis_name="c", subcore_axis_name="s")

    @functools.partial(
        pl.kernel,
        mesh=mesh,
        out_type=jax.ShapeDtypeStruct((B, D), jnp.float32),
        scratch_types=[
            pltpu.VMEM((b_per_w,), jnp.int32),
            pltpu.VMEM((b_per_w, D), jnp.float32),
            pltpu.SemaphoreType.DMA,
        ],
    )
    def gather_kernel(table_hbm, idx_hbm, out_hbm, idx_v, rows_v, sem):
        wid = lax.axis_index("s") * _NC + lax.axis_index("c")
        base = wid * b_per_w
        pltpu.sync_copy(idx_hbm.at[pl.ds(base, b_per_w)], idx_v)
        pltpu.async_copy(table_hbm.at[idx_v], rows_v, sem).wait()
        pltpu.sync_copy(rows_v, out_hbm.at[pl.ds(base, b_per_w)])

    return gather_kernel(table, ids)


def _assemble_t_body(t0i_ref, tf_ref, t0f_ref, az_ref, el_ref, y_ref, x_ref,
                     emb_ref, gsp_ref, o_ref):
    bl = o_ref.shape[2]
    o_ref[:, 0:8, :] = tf_ref[...]
    o_ref[:, 8:16, :] = jnp.broadcast_to(t0f_ref[...][None], (T, D_TIME, bl))
    o_ref[:, 16:17, :] = jnp.reshape(az_ref[...], (T, 1, bl))
    o_ref[:, 17:18, :] = jnp.reshape(el_ref[...], (T, 1, bl))
    o_ref[:, 18:26, :] = jnp.broadcast_to(y_ref[...][None], (T, D_SPACE, bl))
    o_ref[:, 26:34, :] = jnp.broadcast_to(x_ref[...][None], (T, D_SPACE, bl))
    o_ref[:, 34:51, :] = jnp.zeros((T, SPACER, bl), jnp.float32)
    o_ref[:, 51:52, :] = jnp.ones((T, 1, bl), jnp.float32)
    o_ref[:, 52:68, :] = jnp.broadcast_to(emb_ref[...][None],
                                          (T, EMBED_DIM, bl))
    t = lax.broadcasted_iota(jnp.int32, (T, 1, bl), 0)
    o_ref[:, 68:69, :] = jnp.where(t <= t0i_ref[0], gsp_ref[...], 0.0)


def _assemble_t(t0i, tf_t, t0f_t, az_t, el_t, y_t, x_t, emb_t, gsp_t,
                block_l=512):
    B = tf_t.shape[2]
    grid = (B // block_l,)

    def m3(i):
        return (0, 0, i)

    def m2(i):
        return (0, i)

    return pl.pallas_call(
        _assemble_t_body,
        grid=grid,
        in_specs=[
            pl.BlockSpec(memory_space=pltpu.SMEM),
            pl.BlockSpec((T, D_TIME, block_l), m3),
            pl.BlockSpec((D_TIME, block_l), m2),
            pl.BlockSpec((T, block_l), m2),
            pl.BlockSpec((T, block_l), m2),
            pl.BlockSpec((D_SPACE, block_l), m2),
            pl.BlockSpec((D_SPACE, block_l), m2),
            pl.BlockSpec((EMBED_DIM, block_l), m2),
            pl.BlockSpec((T, 1, block_l), m3),
        ],
        out_specs=pl.BlockSpec((T, D_OUT, block_l), m3),
        out_shape=jax.ShapeDtypeStruct((T, D_OUT, B), jnp.float32),
    )(t0i, tf_t, t0f_t, az_t, el_t, y_t, x_t, emb_t, gsp_t)


def kernel(gsp, gsp_time_utc_fourier, gsp_solar_azimuth, gsp_solar_elevation,
           gsp_y_osgb_fourier, gsp_x_osgb_fourier, gsp_time_utc_fourier_t0,
           embedding_table, gsp_id, t0_idx):
    B = gsp.shape[0]
    ids = jnp.reshape(gsp_id, (B,))
    table_padded = jnp.pad(embedding_table,
                           ((0, 0), (0, 128 - embedding_table.shape[1])))
    emb = _sc_embedding_gather(table_padded, ids)
    emb_t = jnp.transpose(emb[:, 0:EMBED_DIM], (1, 0))
    t0i = jnp.reshape(jnp.asarray(t0_idx, jnp.int32), (1,))
    out_t = _assemble_t(
        t0i,
        jnp.transpose(gsp_time_utc_fourier, (1, 2, 0)),
        jnp.transpose(gsp_time_utc_fourier_t0, (1, 0)),
        jnp.transpose(gsp_solar_azimuth, (1, 0)),
        jnp.transpose(gsp_solar_elevation, (1, 0)),
        jnp.reshape(gsp_y_osgb_fourier, (B, D_SPACE)).T,
        jnp.reshape(gsp_x_osgb_fourier, (B, D_SPACE)).T,
        emb_t,
        jnp.transpose(gsp, (1, 2, 0)),
    )
    return jnp.transpose(out_t, (2, 0, 1))
